# baseline (speedup 1.0000x reference)
import functools

import jax
import jax.numpy as jnp
from jax import lax
from jax.experimental import pallas as pl
from jax.experimental.pallas import tpu as pltpu
from jax.experimental.pallas import tpu_sc as plsc

_B, _T, _D = 4, 2048, 768
_ROWS = _B * _T
_NW = 32
_RPW = _ROWS // _NW
_CH = 64
_NCH = _RPW // _CH
_LANES = 16
_DC = _D // _LANES
_EPS = 1e-12


def _rsqrt(v):
    i = lax.bitcast_convert_type(v, jnp.int32)
    i = jnp.int32(0x5F3759DF) - lax.shift_right_logical(i, 1)
    y = lax.bitcast_convert_type(i, jnp.float32)
    for _ in range(3):
        y = y * (1.5 - 0.5 * v * y * y)
    return y


def _sc_body(x_hbm, seg_hbm, tok_hbm, pos_hbm, segemb_hbm, gamma_hbm,
             beta_hbm, out_hbm, idx_v, sid_v, segt_v, gam_v, bet_v,
             tok_v, pos_v, sem):
    cid = lax.axis_index("c")
    sid = lax.axis_index("s")
    wid = sid * 2 + cid
    base = wid * _RPW

    pltpu.sync_copy(segemb_hbm, segt_v)
    pltpu.sync_copy(gamma_hbm, gam_v)
    pltpu.sync_copy(beta_hbm, bet_v)

    zeros = jnp.zeros((_LANES,), jnp.float32)

    for ch in range(_NCH):
        rbase = base + ch * _CH
        t0 = rbase % _T
        pltpu.sync_copy(x_hbm.at[pl.ds(rbase, _CH)], idx_v)
        pltpu.sync_copy(seg_hbm.at[pl.ds(rbase, _CH)], sid_v.at[pl.ds(0, _CH)])
        pltpu.sync_copy(pos_hbm.at[pl.ds(t0, _CH)], pos_v)
        pltpu.async_copy(tok_hbm.at[idx_v], tok_v, sem).wait()

        def row_body(r, _):
            s_id = sid_v[pl.ds(r, _LANES)][0]

            def acc_body(c, carry):
                vs, vq = carry
                off = pl.ds(c * _LANES, _LANES)
                h = tok_v[r, off] + pos_v[r, off] + segt_v[s_id, off]
                tok_v[r, off] = h
                return (vs + h, vq + h * h)

            vs, vq = lax.fori_loop(0, _DC, acc_body, (zeros, zeros))
            mu = jnp.sum(vs) * (1.0 / _D)
            var = jnp.sum(vq) * (1.0 / _D) - mu * mu
            rinv = _rsqrt(var + _EPS)

            def norm_body(c, carry):
                off = pl.ds(c * _LANES, _LANES)
                h = tok_v[r, off]
                tok_v[r, off] = (h - mu) * rinv * gam_v[off] + bet_v[off]
                return carry

            return lax.fori_loop(0, _DC, norm_body, _)

        lax.fori_loop(0, _CH, row_body, 0)
        pltpu.sync_copy(tok_v, out_hbm.at[pl.ds(rbase, _CH)])


@jax.jit
def _emb_ln(xf, sf, tok_emb, pos_emb, seg_emb, gamma, beta):
    mesh = plsc.VectorSubcoreMesh(core_axis_name="c", subcore_axis_name="s")
    call = functools.partial(
        pl.kernel,
        mesh=mesh,
        out_type=jax.ShapeDtypeStruct((_ROWS, _D), jnp.float32),
        compiler_params=pltpu.CompilerParams(needs_layout_passes=False),
        scratch_types=[
            pltpu.VMEM((_CH,), jnp.int32),
            pltpu.VMEM((_CH + _LANES,), jnp.int32),
            pltpu.VMEM((2, _D), jnp.float32),
            pltpu.VMEM((_D,), jnp.float32),
            pltpu.VMEM((_D,), jnp.float32),
            pltpu.VMEM((_CH, _D), jnp.float32),
            pltpu.VMEM((_CH, _D), jnp.float32),
            pltpu.SemaphoreType.DMA,
        ],
    )(_sc_body)
    return call(xf, sf, tok_emb, pos_emb, seg_emb, gamma, beta)


def kernel(x, segments, tok_emb, pos_emb, seg_emb, gamma, beta):
    xf = x.reshape(-1)
    sf = segments.reshape(-1)
    out = _emb_ln(xf, sf, tok_emb, pos_emb, seg_emb, gamma, beta)
    return out.reshape(_B, _T, _D)


# speedup vs baseline: 1.2353x; 1.2353x over previous
import functools

import jax
import jax.numpy as jnp
from jax import lax
from jax.experimental import pallas as pl
from jax.experimental.pallas import tpu as pltpu
from jax.experimental.pallas import tpu_sc as plsc

_B, _T, _D = 4, 2048, 768
_ROWS = _B * _T
_NW = 32
_RPW = _ROWS // _NW
_CH = 32
_NCH = _RPW // _CH
_LANES = 16
_DC = _D // _LANES
_EPS = 1e-12


def _rsqrt(v):
    i = lax.bitcast_convert_type(v, jnp.int32)
    i = jnp.int32(0x5F3759DF) - lax.shift_right_logical(i, 1)
    y = lax.bitcast_convert_type(i, jnp.float32)
    for _ in range(3):
        y = y * (1.5 - 0.5 * v * y * y)
    return y


def _sc_body(x_hbm, seg_hbm, tok_hbm, pos_hbm, segemb_hbm, gamma_hbm,
             beta_hbm, out_hbm, idx_v, sid_v, segt_v, gam_v, bet_v,
             tok_v, pos_v, gsem0, gsem1, wsem0, wsem1):
    cid = lax.axis_index("c")
    sid = lax.axis_index("s")
    wid = sid * 2 + cid
    base = wid * _RPW
    gsems = (gsem0, gsem1)
    wsems = (wsem0, wsem1)

    pltpu.sync_copy(segemb_hbm, segt_v)
    pltpu.sync_copy(gamma_hbm, gam_v)
    pltpu.sync_copy(beta_hbm, bet_v)

    zeros = jnp.zeros((_LANES,), jnp.float32)

    def fetch(c, buf):
        rbase = base + c * _CH
        t0 = rbase % _T
        pltpu.sync_copy(x_hbm.at[pl.ds(rbase, _CH)], idx_v.at[buf])
        pltpu.sync_copy(seg_hbm.at[pl.ds(rbase, _CH)],
                        sid_v.at[buf, pl.ds(0, _CH)])
        pltpu.sync_copy(pos_hbm.at[pl.ds(t0, _CH)], pos_v.at[buf])
        return pltpu.async_copy(tok_hbm.at[idx_v.at[buf]], tok_v.at[buf],
                                gsems[buf])

    gdesc = {0: fetch(0, 0)}
    wdesc = {}

    for c in range(_NCH):
        b = c % 2
        if c + 1 < _NCH:
            nb = (c + 1) % 2
            if c >= 1:
                wdesc[c - 1].wait()
            gdesc[c + 1] = fetch(c + 1, nb)
        gdesc[c].wait()

        def row_body(r, _):
            s_id = sid_v[b, pl.ds(r, _LANES)][0]

            def acc_body(cc, carry):
                vs, vq = carry
                off = pl.ds(cc * _LANES, _LANES)
                h = tok_v[b, r, off] + pos_v[b, r, off] + segt_v[s_id, off]
                tok_v[b, r, off] = h
                return (vs + h, vq + h * h)

            vs, vq = lax.fori_loop(0, _DC, acc_body, (zeros, zeros),
                                   unroll=8)
            mu = jnp.sum(vs) * (1.0 / _D)
            var = jnp.sum(vq) * (1.0 / _D) - mu * mu
            rinv = _rsqrt(var + _EPS)

            def norm_body(cc, carry):
                off = pl.ds(cc * _LANES, _LANES)
                h = tok_v[b, r, off]
                tok_v[b, r, off] = (h - mu) * rinv * gam_v[off] + bet_v[off]
                return carry

            return lax.fori_loop(0, _DC, norm_body, _, unroll=8)

        lax.fori_loop(0, _CH, row_body, 0)
        rbase = base + c * _CH
        wdesc[c] = pltpu.async_copy(tok_v.at[b],
                                    out_hbm.at[pl.ds(rbase, _CH)], wsems[b])

    wdesc[_NCH - 2].wait()
    wdesc[_NCH - 1].wait()


@jax.jit
def _emb_ln(xf, sf, tok_emb, pos_emb, seg_emb, gamma, beta):
    mesh = plsc.VectorSubcoreMesh(core_axis_name="c", subcore_axis_name="s")
    call = functools.partial(
        pl.kernel,
        mesh=mesh,
        out_type=jax.ShapeDtypeStruct((_ROWS, _D), jnp.float32),
        compiler_params=pltpu.CompilerParams(needs_layout_passes=False),
        scratch_types=[
            pltpu.VMEM((2, _CH), jnp.int32),
            pltpu.VMEM((2, _CH + _LANES), jnp.int32),
            pltpu.VMEM((2, _D), jnp.float32),
            pltpu.VMEM((_D,), jnp.float32),
            pltpu.VMEM((_D,), jnp.float32),
            pltpu.VMEM((2, _CH, _D), jnp.float32),
            pltpu.VMEM((2, _CH, _D), jnp.float32),
            pltpu.SemaphoreType.DMA,
            pltpu.SemaphoreType.DMA,
            pltpu.SemaphoreType.DMA,
            pltpu.SemaphoreType.DMA,
        ],
    )(_sc_body)
    return call(xf, sf, tok_emb, pos_emb, seg_emb, gamma, beta)


def kernel(x, segments, tok_emb, pos_emb, seg_emb, gamma, beta):
    xf = x.reshape(-1)
    sf = segments.reshape(-1)
    out = _emb_ln(xf, sf, tok_emb, pos_emb, seg_emb, gamma, beta)
    return out.reshape(_B, _T, _D)


# speedup vs baseline: 1.5755x; 1.2754x over previous
import functools

import jax
import jax.numpy as jnp
from jax import lax
from jax.experimental import pallas as pl
from jax.experimental.pallas import tpu as pltpu
from jax.experimental.pallas import tpu_sc as plsc

_B, _T, _D = 4, 2048, 768
_ROWS = _B * _T
_NW = 32
_RPW = _ROWS // _NW
_CH = 32
_NCH = _RPW // _CH
_LANES = 16
_DC = _D // _LANES
_NCG = 4
_CGW = _DC // _NCG
_EPS = 1e-12


def _rsqrt(v):
    i = lax.bitcast_convert_type(v, jnp.int32)
    i = jnp.int32(0x5F3759DF) - lax.shift_right_logical(i, 1)
    y = lax.bitcast_convert_type(i, jnp.float32)
    for _ in range(3):
        y = y * (1.5 - 0.5 * v * y * y)
    return y


def _sc_body(x_hbm, seg_hbm, tok_hbm, pos_hbm, segemb_hbm, gamma_hbm,
             beta_hbm, out_hbm, idx_v, sid_v, segt_v, gam_v, bet_v,
             tok_v, pos_v, svs_v, svq_v, ab_v, bb_v, gsem0, gsem1,
             wsem0, wsem1):
    cid = lax.axis_index("c")
    sid = lax.axis_index("s")
    wid = sid * 2 + cid
    base = wid * _RPW
    gsems = (gsem0, gsem1)
    wsems = (wsem0, wsem1)

    pltpu.sync_copy(segemb_hbm, segt_v)
    pltpu.sync_copy(gamma_hbm, gam_v)
    pltpu.sync_copy(beta_hbm, bet_v)

    zeros = jnp.zeros((_LANES,), jnp.float32)
    iota = lax.iota(jnp.int32, _LANES)

    def fetch(c, buf):
        rbase = base + c * _CH
        t0 = rbase % _T
        pltpu.sync_copy(x_hbm.at[pl.ds(rbase, _CH)], idx_v.at[buf])
        pltpu.sync_copy(seg_hbm.at[pl.ds(rbase, _CH)],
                        sid_v.at[buf, pl.ds(0, _CH)])
        pltpu.sync_copy(pos_hbm.at[pl.ds(t0, _CH)], pos_v.at[buf])
        return pltpu.async_copy(tok_hbm.at[idx_v.at[buf]], tok_v.at[buf],
                                gsems[buf])

    gdesc = {0: fetch(0, 0)}
    wdesc = {}

    for c in range(_NCH):
        b = c % 2
        if c + 1 < _NCH:
            nb = (c + 1) % 2
            if c >= 1:
                wdesc[c - 1].wait()
            gdesc[c + 1] = fetch(c + 1, nb)
        gdesc[c].wait()

        def row_body(r, _):
            s_id = sid_v[b, pl.ds(r, _LANES)][0]

            def acc_body(cc, carry):
                vs, vq = carry
                off = pl.ds(cc * _LANES, _LANES)
                h = tok_v[b, r, off] + pos_v[b, r, off] + segt_v[s_id, off]
                tok_v[b, r, off] = h
                return (vs + h, vq + h * h)

            vs, vq = lax.fori_loop(0, _DC, acc_body, (zeros, zeros),
                                   unroll=8)
            svs_v[r] = vs
            svq_v[r] = vq
            return _

        lax.fori_loop(0, _CH, row_body, 0)

        for g in range(_CH // _LANES):
            rows = g * _LANES + iota
            tsum = zeros
            tsq = zeros
            for l in range(_LANES):
                col = jnp.full((_LANES,), l, jnp.int32)
                tsum = tsum + plsc.load_gather(svs_v, [rows, col])
                tsq = tsq + plsc.load_gather(svq_v, [rows, col])
            mu = tsum * (1.0 / _D)
            var = tsq * (1.0 / _D) - mu * mu
            y = _rsqrt(var + _EPS)
            ab_v[pl.ds(g * _LANES, _LANES)] = y
            bb_v[pl.ds(g * _LANES, _LANES)] = -mu * y

        for cg in range(_NCG):
            gs = [gam_v[pl.ds((cg * _CGW + j) * _LANES, _LANES)]
                  for j in range(_CGW)]
            bs = [bet_v[pl.ds((cg * _CGW + j) * _LANES, _LANES)]
                  for j in range(_CGW)]

            def nrow_body(r, _, gs=gs, bs=bs, cg=cg):
                a = ab_v[pl.ds(r, _LANES)][0]
                bb = bb_v[pl.ds(r, _LANES)][0]
                for j in range(_CGW):
                    off = pl.ds((cg * _CGW + j) * _LANES, _LANES)
                    h = tok_v[b, r, off]
                    tok_v[b, r, off] = (h * a + bb) * gs[j] + bs[j]
                return _

            lax.fori_loop(0, _CH, nrow_body, 0)

        rbase = base + c * _CH
        wdesc[c] = pltpu.async_copy(tok_v.at[b],
                                    out_hbm.at[pl.ds(rbase, _CH)], wsems[b])

    wdesc[_NCH - 2].wait()
    wdesc[_NCH - 1].wait()


@jax.jit
def _emb_ln(xf, sf, tok_emb, pos_emb, seg_emb, gamma, beta):
    mesh = plsc.VectorSubcoreMesh(core_axis_name="c", subcore_axis_name="s")
    call = functools.partial(
        pl.kernel,
        mesh=mesh,
        out_type=jax.ShapeDtypeStruct((_ROWS, _D), jnp.float32),
        compiler_params=pltpu.CompilerParams(needs_layout_passes=False),
        scratch_types=[
            pltpu.VMEM((2, _CH), jnp.int32),
            pltpu.VMEM((2, _CH + _LANES), jnp.int32),
            pltpu.VMEM((2, _D), jnp.float32),
            pltpu.VMEM((_D,), jnp.float32),
            pltpu.VMEM((_D,), jnp.float32),
            pltpu.VMEM((2, _CH, _D), jnp.float32),
            pltpu.VMEM((2, _CH, _D), jnp.float32),
            pltpu.VMEM((_CH, _LANES), jnp.float32),
            pltpu.VMEM((_CH, _LANES), jnp.float32),
            pltpu.VMEM((_CH + _LANES,), jnp.float32),
            pltpu.VMEM((_CH + _LANES,), jnp.float32),
            pltpu.SemaphoreType.DMA,
            pltpu.SemaphoreType.DMA,
            pltpu.SemaphoreType.DMA,
            pltpu.SemaphoreType.DMA,
        ],
    )(_sc_body)
    return call(xf, sf, tok_emb, pos_emb, seg_emb, gamma, beta)


def kernel(x, segments, tok_emb, pos_emb, seg_emb, gamma, beta):
    xf = x.reshape(-1)
    sf = segments.reshape(-1)
    out = _emb_ln(xf, sf, tok_emb, pos_emb, seg_emb, gamma, beta)
    return out.reshape(_B, _T, _D)
